# Initial kernel scaffold; baseline (speedup 1.0000x reference)
import jax
import jax.numpy as jnp
from jax.experimental import pallas as pl


def kernel(x, emb_table, W_ih, W_hh, b_ih, b_hh, W_dense, b_dense):
    raise NotImplementedError("write your pallas kernel here")


# speedup vs baseline: 3.5274x; 3.5274x over previous
import functools

import jax
import jax.numpy as jnp
from jax import lax
from jax.experimental import pallas as pl
from jax.experimental.pallas import tpu as pltpu
from jax.experimental.pallas import tpu_sc as plsc

VOCAB = 30000
EMB = 125
HID = 100
OUT = 2
B = 1024
L = 200

DPAD = 128
HPAD = 128
G3 = 3 * HPAD

NC = 2
NS = 16
NW = NC * NS
ROWS = L * B
RPW = ROWS // NW
CH = 128
NCH = RPW // CH


def _sc_gather(table, idx):
    mesh = plsc.VectorSubcoreMesh(core_axis_name="c", subcore_axis_name="s")

    @functools.partial(
        pl.kernel,
        mesh=mesh,
        out_type=jax.ShapeDtypeStruct((ROWS, DPAD), jnp.float32),
        scratch_types=[
            pltpu.VMEM((NCH, CH), jnp.int32),
            pltpu.VMEM((CH, DPAD), jnp.float32),
            pltpu.SemaphoreType.DMA,
        ],
    )
    def gather_kernel(table_hbm, idx_hbm, out_hbm, idx_v, buf, sem):
        wid = lax.axis_index("s") * NC + lax.axis_index("c")
        base = wid * RPW
        pltpu.sync_copy(idx_hbm.at[wid], idx_v)

        def body(c, carry):
            pltpu.async_copy(table_hbm.at[idx_v.at[c]], buf, sem).wait()
            pltpu.sync_copy(buf, out_hbm.at[pl.ds(base + c * CH, CH)])
            return carry

        lax.fori_loop(0, NCH, body, 0)

    return gather_kernel(table, idx)


def _gru_scan_body(e_ref, wih_ref, whh_ref, bih_ref, bhh_ref, wd_ref, bd_ref,
                   out_ref, h_ref):
    t = pl.program_id(0)

    @pl.when(t == 0)
    def _():
        h_ref[...] = jnp.zeros_like(h_ref)

    h = h_ref[...]
    e_t = e_ref[0]
    gi = jnp.dot(e_t, wih_ref[...], preferred_element_type=jnp.float32)
    gi = gi + bih_ref[...]
    gh = jnp.dot(h, whh_ref[...], preferred_element_type=jnp.float32)
    gh = gh + bhh_ref[...]
    r = jax.nn.sigmoid(gi[:, :HPAD] + gh[:, :HPAD])
    z = jax.nn.sigmoid(gi[:, HPAD:2 * HPAD] + gh[:, HPAD:2 * HPAD])
    n = jnp.tanh(gi[:, 2 * HPAD:] + r * gh[:, 2 * HPAD:])
    h_new = (1.0 - z) * n + z * h
    h_ref[...] = h_new

    @pl.when(t == L - 1)
    def _():
        a = jnp.tanh(h_new)
        logits = jnp.dot(a, wd_ref[...], preferred_element_type=jnp.float32)
        logits = logits + bd_ref[...]
        m = jnp.max(logits, axis=-1, keepdims=True)
        p = jnp.exp(logits - m)
        p = p / jnp.sum(p, axis=-1, keepdims=True)
        out_ref[...] = p[:, :OUT]


def _gru_scan(e, wih, whh, bih, bhh, wd, bd):
    return pl.pallas_call(
        _gru_scan_body,
        grid=(L,),
        in_specs=[
            pl.BlockSpec((1, B, DPAD), lambda t: (t, 0, 0)),
            pl.BlockSpec((DPAD, G3), lambda t: (0, 0)),
            pl.BlockSpec((HPAD, G3), lambda t: (0, 0)),
            pl.BlockSpec((1, G3), lambda t: (0, 0)),
            pl.BlockSpec((1, G3), lambda t: (0, 0)),
            pl.BlockSpec((HPAD, HPAD), lambda t: (0, 0)),
            pl.BlockSpec((1, HPAD), lambda t: (0, 0)),
        ],
        out_specs=pl.BlockSpec((B, OUT), lambda t: (0, 0)),
        out_shape=jax.ShapeDtypeStruct((B, OUT), jnp.float32),
        scratch_shapes=[pltpu.VMEM((B, HPAD), jnp.float32)],
    )(e, wih, whh, bih, bhh, wd, bd)


def _pad_gates_2d(w, rows_to):
    rows = w.shape[0]
    parts = []
    for g in range(3):
        wg = w[:, g * HID:(g + 1) * HID]
        parts.append(jnp.pad(wg, ((0, rows_to - rows), (0, HPAD - HID))))
    return jnp.concatenate(parts, axis=1)


def _pad_gates_1d(b):
    parts = [jnp.pad(b[g * HID:(g + 1) * HID], (0, HPAD - HID))
             for g in range(3)]
    return jnp.concatenate(parts)[None, :]


def kernel(x, emb_table, W_ih, W_hh, b_ih, b_hh, W_dense, b_dense):
    idx = x.astype(jnp.int32).T.reshape(NW, NCH, CH)
    table = jnp.pad(emb_table, ((0, 0), (0, DPAD - EMB)))
    e = _sc_gather(table, idx).reshape(L, B, DPAD)

    wih = _pad_gates_2d(W_ih, DPAD)
    whh = _pad_gates_2d(W_hh, HPAD)
    bih = _pad_gates_1d(b_ih)
    bhh = _pad_gates_1d(b_hh)
    wd = jnp.pad(W_dense.T, ((0, HPAD - HID), (0, HPAD - OUT)))
    bd = jnp.pad(b_dense, (0, HPAD - OUT), constant_values=-1e30)[None, :]

    return _gru_scan(e, wih, whh, bih, bhh, wd, bd)
